# baseline (speedup 1.0000x reference)
import jax
import jax.numpy as jnp
from jax.experimental import pallas as pl
from jax.experimental.pallas import tpu as pltpu

_FB = 256
_NBUF = 6


def _blend_body(y_ref, t_ref, x_hbm, o_hbm, ibuf, obuf, isem, osem):
    i = pl.program_id(0)
    F, B = x_hbm.shape
    n = F // _FB
    slot = jax.lax.rem(i, _NBUF)

    def in_copy(step, buf):
        return pltpu.make_async_copy(
            x_hbm.at[pl.ds(step * _FB, _FB)], ibuf.at[buf], isem.at[buf])

    def out_copy(step, buf):
        return pltpu.make_async_copy(
            obuf.at[buf], o_hbm.at[pl.ds(step * _FB, _FB)], osem.at[buf])

    @pl.when(i == 0)
    def _prologue():
        for k in range(min(_NBUF, n)):
            in_copy(k, k).start()

    in_copy(i, slot).wait()

    @pl.when(i >= _NBUF)
    def _wait_prev_out():
        out_copy(i - _NBUF, slot).wait()

    m = (y_ref[...] == 0).astype(jnp.float32)
    t = t_ref[:, :1]
    xv = ibuf[slot]
    obuf[slot] = xv - ((xv + 0.75) * m) * t

    out_copy(i, slot).start()

    @pl.when(i + _NBUF < n)
    def _next_in():
        in_copy(i + _NBUF, slot).start()

    @pl.when(i == n - 1)
    def _epilogue():
        for k in range(max(n - _NBUF, 0), n):
            out_copy(k, k % _NBUF).wait()


def kernel(x, y, template):
    B, C, S, _ = x.shape
    F = C * S * S
    xt = x.transpose(1, 2, 3, 0).reshape(F, B)
    yt = y.reshape(1, B)
    tcol = jnp.broadcast_to(template.reshape(S * S, 1), (S * S, 128))
    nper = (S * S) // _FB
    out = pl.pallas_call(
        _blend_body,
        grid=(F // _FB,),
        in_specs=[
            pl.BlockSpec((1, B), lambda i: (0, 0)),
            pl.BlockSpec((_FB, 128), lambda i: (i % nper, 0)),
            pl.BlockSpec(memory_space=pltpu.MemorySpace.HBM),
        ],
        out_specs=pl.BlockSpec(memory_space=pltpu.MemorySpace.HBM),
        out_shape=jax.ShapeDtypeStruct((F, B), x.dtype),
        scratch_shapes=[
            pltpu.VMEM((_NBUF, _FB, B), jnp.float32),
            pltpu.VMEM((_NBUF, _FB, B), jnp.float32),
            pltpu.SemaphoreType.DMA((_NBUF,)),
            pltpu.SemaphoreType.DMA((_NBUF,)),
        ],
    )(yt, tcol, xt)
    return (out.reshape(C, S, S, B).transpose(3, 0, 1, 2), y)


# speedup vs baseline: 1.0212x; 1.0212x over previous
import jax
import jax.numpy as jnp
from jax.experimental import pallas as pl

_FB = 512


def _blend_body(y_ref, t_ref, x_ref, o_ref):
    i = pl.program_id(0)
    nper = t_ref.shape[0] // _FB
    m = (y_ref[...] == 0).astype(jnp.float32)
    t = t_ref[pl.ds(jax.lax.rem(i, nper) * _FB, _FB), :1]
    xv = x_ref[...]
    o_ref[...] = xv - ((xv + 0.75) * m) * t


def kernel(x, y, template):
    B, C, S, _ = x.shape
    F = C * S * S
    xt = x.transpose(1, 2, 3, 0).reshape(F, B)
    yt = y.reshape(1, B)
    tcol = jnp.broadcast_to(template.reshape(S * S, 1), (S * S, 128))
    out = pl.pallas_call(
        _blend_body,
        grid=(F // _FB,),
        in_specs=[
            pl.BlockSpec((1, B), lambda i: (0, 0)),
            pl.BlockSpec((S * S, 128), lambda i: (0, 0)),
            pl.BlockSpec((_FB, B), lambda i: (i, 0)),
        ],
        out_specs=pl.BlockSpec((_FB, B), lambda i: (i, 0)),
        out_shape=jax.ShapeDtypeStruct((F, B), x.dtype),
    )(yt, tcol, xt)
    return (out.reshape(C, S, S, B).transpose(3, 0, 1, 2), y)
